# baseline (speedup 1.0000x reference)
import jax
import jax.numpy as jnp
from jax import lax
from jax.experimental import pallas as pl
from jax.experimental.pallas import tpu as pltpu
from jax.experimental.pallas import tpu_sc as plsc

N = 10000
E = 320000
G = 128
FW = 128
NC = 2
NS = 16
NW = NC * NS
CH = 128
NCH = 80
NCHH = NCH // 2
EPW = NCH * CH
EPAD = NW * EPW
NPAD = 10240
RPT = NPAD // NS


def _sc_mesh():
    return plsc.VectorSubcoreMesh(core_axis_name="c", subcore_axis_name="s",
                                  num_cores=NC, num_subcores=NS)


def _make_agg(FA):
    NSUB = FA // 16

    def body(y_hbm, src_hbm, dst_hbm, out_hbm,
             src_v, dst_v, rows_a, rows_b, wout, tbl, acc,
             sem_a, sem_b, sem_sa, sem_sb):
        c = lax.axis_index("c")
        s = lax.axis_index("s")
        wid = s * NC + c
        zeros16 = jnp.zeros((16,), jnp.float32)

        def zstep(r4, carry):
            r = r4 * 4
            for d in range(4):
                for k in range(NSUB):
                    rows_a[r + d, pl.ds(k * 16, 16)] = zeros16
            return carry

        lax.fori_loop(0, CH // 4, zstep, 0)
        for t in range(RPT // CH):
            pltpu.sync_copy(rows_a, acc.at[pl.ds(s * RPT + t * CH, CH)])
        for t in range(RPT // CH):
            pltpu.sync_copy(y_hbm.at[pl.ds(s * RPT + t * CH, CH)], wout)

            def tstep(r4, carry):
                r = r4 * 4
                for d in range(4):
                    for k in range(NSUB):
                        rows_a[r + d, pl.ds(k * 16, 16)] = (
                            wout[r + d, pl.ds(k * 16, 16)])
                return carry

            lax.fori_loop(0, CH // 4, tstep, 0)
            pltpu.sync_copy(rows_a, tbl.at[pl.ds(s * RPT + t * CH, CH)])
        plsc.subcore_barrier()

        def fire_g(j, rows, sem):
            pltpu.async_copy(tbl.at[src_v.at[j]], rows, sem)

        def wait_g(j, rows, sem):
            pltpu.make_async_copy(tbl.at[src_v.at[j]], rows, sem).wait()

        def fire_s(j, rows, sem):
            pltpu.async_copy(rows, acc.at[dst_v.at[j]], sem, add=True)

        def wait_s(j, rows, sem):
            pltpu.make_async_copy(rows, acc.at[dst_v.at[j]], sem).wait()

        def step(j2, carry):
            j = j2 * 2
            wait_g(j, rows_a, sem_a)
            fire_s(j, rows_a, sem_sa)
            wait_g(j + 1, rows_b, sem_b)
            fire_s(j + 1, rows_b, sem_sb)
            wait_s(j, rows_a, sem_sa)

            @pl.when(j + 2 < NCHH)
            def _():
                fire_g(j + 2, rows_a, sem_a)

            wait_s(j + 1, rows_b, sem_sb)

            @pl.when(j + 3 < NCHH)
            def _():
                fire_g(j + 3, rows_b, sem_b)

            return carry

        for half in range(2):
            pltpu.sync_copy(src_hbm.at[wid, pl.ds(half * NCHH, NCHH)], src_v)
            pltpu.sync_copy(dst_hbm.at[wid, pl.ds(half * NCHH, NCHH)], dst_v)
            fire_g(0, rows_a, sem_a)
            fire_g(1, rows_b, sem_b)
            lax.fori_loop(0, NCHH // 2, step, 0)

        plsc.subcore_barrier()
        for t in range(RPT // CH):
            pltpu.sync_copy(acc.at[pl.ds(s * RPT + t * CH, CH)], rows_a)

            def wstep(r4, carry):
                r = r4 * 4
                for d in range(4):
                    for k in range(NSUB):
                        wout[r + d, pl.ds(k * 16, 16)] = (
                            rows_a[r + d, pl.ds(k * 16, 16)])
                return carry

            lax.fori_loop(0, CH // 4, wstep, 0)
            pltpu.sync_copy(wout, out_hbm.at[c, pl.ds(s * RPT + t * CH, CH)])

    return pl.kernel(
        body,
        out_type=jax.ShapeDtypeStruct((NC, NPAD, FW), jnp.float32),
        mesh=_sc_mesh(),
        compiler_params=pltpu.CompilerParams(use_tc_tiling_on_sc=False),
        scratch_types=[
            pltpu.VMEM((NCHH, CH), jnp.int32),
            pltpu.VMEM((NCHH, CH), jnp.int32),
            pltpu.VMEM((CH, FA), jnp.float32),
            pltpu.VMEM((CH, FA), jnp.float32),
            pltpu.VMEM((CH, FW), jnp.float32),
            pltpu.VMEM_SHARED((NPAD, FA), jnp.float32),
            pltpu.VMEM_SHARED((NPAD, FA), jnp.float32),
            pltpu.SemaphoreType.DMA,
            pltpu.SemaphoreType.DMA,
            pltpu.SemaphoreType.DMA,
            pltpu.SemaphoreType.DMA,
        ],
    )


def _deg_body(dst_hbm, out_hbm, dst_v, ones_v, rows, pk, acc, sem_s):
    c = lax.axis_index("c")
    s = lax.axis_index("s")
    wid = s * NC + c
    zeros16 = jnp.zeros((16,), jnp.float32)
    ones16 = jnp.ones((16,), jnp.float32)

    def fill(r, carry):
        ones_v[r, pl.ds(0, 16)] = ones16
        pk[r, pl.ds(0, 16)] = zeros16
        return carry

    lax.fori_loop(0, CH, fill, 0)
    for t in range(RPT // CH):
        pltpu.sync_copy(pk, acc.at[pl.ds(s * RPT + t * CH, CH)])
    pltpu.sync_copy(dst_hbm.at[wid], dst_v)
    plsc.subcore_barrier()

    DEPTH = 4

    def fire_s(j):
        pltpu.async_copy(ones_v, acc.at[dst_v.at[j]], sem_s, add=True)

    def wait_s(j):
        pltpu.make_async_copy(ones_v, acc.at[dst_v.at[j]], sem_s).wait()

    for j in range(DEPTH):
        fire_s(j)

    def step(j, carry):
        fire_s(j)
        wait_s(j)
        return carry

    lax.fori_loop(DEPTH, NCH, step, 0)
    for j in range(DEPTH):
        wait_s(j)
    plsc.subcore_barrier()
    for t in range(RPT // CH):
        pltpu.sync_copy(acc.at[pl.ds(s * RPT + t * CH, CH)], pk)

        def wstep(r4, carry):
            r = r4 * 4
            for d in range(4):
                rows[r + d, pl.ds(0, 16)] = pk[r + d, pl.ds(0, 16)]
            return carry

        lax.fori_loop(0, CH // 4, wstep, 0)
        pltpu.sync_copy(rows, out_hbm.at[c, pl.ds(s * RPT + t * CH, CH)])


_deg = pl.kernel(
    _deg_body,
    out_type=jax.ShapeDtypeStruct((NC, NPAD, FW), jnp.float32),
    mesh=_sc_mesh(),
    compiler_params=pltpu.CompilerParams(use_tc_tiling_on_sc=False),
    scratch_types=[
        pltpu.VMEM((NCH, CH), jnp.int32),
        pltpu.VMEM((CH, 16), jnp.float32),
        pltpu.VMEM((CH, FW), jnp.float32),
        pltpu.VMEM((CH, 16), jnp.float32),
        pltpu.VMEM_SHARED((NPAD, 16), jnp.float32),
        pltpu.SemaphoreType.DMA,
    ],
)

_agg = {FA: _make_agg(FA) for FA in (64, 32, 16)}


def _pad_table(v):
    n, f = v.shape
    if f < FW:
        v = jnp.concatenate(
            [v, jnp.zeros((n, FW - f), jnp.float32)], axis=1)
    if n < NPAD:
        v = jnp.concatenate(
            [v, jnp.zeros((NPAD - n, FW), jnp.float32)], axis=0)
    return v


def _k1_body(x_ref, w_ref, degp_ref, y_ref, dinv_ref):
    deg = degp_ref[0, :N, 0:1] + degp_ref[1, :N, 0:1] + 1.0
    dinv = lax.rsqrt(deg)
    dinv_ref[...] = dinv
    y = dinv * jnp.dot(x_ref[...], w_ref[...],
                       preferred_element_type=jnp.float32)
    y_ref[...] = _pad_table(y)


def _make_kmid(f_in, f_out):
    def body(a_ref, y_ref, dinv_ref, b_ref, w_ref, o_ref):
        dinv = dinv_ref[...]
        pre = dinv * (a_ref[0, :N, :f_in] + a_ref[1, :N, :f_in]
                      + y_ref[:N, :f_in]) + b_ref[...]
        h = jnp.maximum(pre, 0.0)
        y = dinv * jnp.dot(h, w_ref[...], preferred_element_type=jnp.float32)
        o_ref[...] = _pad_table(y)

    return pl.pallas_call(
        body, out_shape=jax.ShapeDtypeStruct((NPAD, FW), jnp.float32))


def _k5_body(a_ref, y_ref, dinv_ref, b_ref, batch_ref, o_ref):
    dinv = dinv_ref[...]
    pre = dinv * (a_ref[0, :N, :2] + a_ref[1, :N, :2]
                  + y_ref[:N, :2]) + b_ref[...]
    oh = (batch_ref[...] == lax.broadcasted_iota(jnp.int32, (1, G), 1))
    oh = oh.astype(jnp.float32)
    cdims = (((0,), (0,)), ((), ()))
    sums = lax.dot_general(oh, pre, cdims, preferred_element_type=jnp.float32)
    cnts = lax.dot_general(oh, jnp.ones((N, 1), jnp.float32), cdims,
                           preferred_element_type=jnp.float32)
    pooled = sums / jnp.maximum(cnts, 1.0)
    m = jnp.max(pooled, axis=1, keepdims=True)
    o_ref[...] = pooled - m - jnp.log(
        jnp.sum(jnp.exp(pooled - m), axis=1, keepdims=True))


_k1 = pl.pallas_call(
    _k1_body,
    out_shape=[jax.ShapeDtypeStruct((NPAD, FW), jnp.float32),
               jax.ShapeDtypeStruct((N, 1), jnp.float32)],
)

_k5 = pl.pallas_call(
    _k5_body,
    out_shape=jax.ShapeDtypeStruct((G, 2), jnp.float32),
)


def kernel(x, edge_index, batch, W1, b1, W2, b2, W3, b3, W4, b4):
    npad = EPAD - E
    pad_src = (jnp.arange(npad, dtype=jnp.int32) * 37) % N
    pad_dst = N + (jnp.arange(npad, dtype=jnp.int32) % (NPAD - N))
    src = jnp.concatenate([edge_index[0], pad_src]).reshape(NW, NCH, CH)
    dst = jnp.concatenate([edge_index[1], pad_dst]).reshape(NW, NCH, CH)

    degp = _deg(dst)
    y1, dinv = _k1(x, W1, degp)
    a1 = _agg[64](y1, src, dst)
    y2 = _make_kmid(64, 64)(a1, y1, dinv, b1.reshape(1, -1), W2)
    a2 = _agg[64](y2, src, dst)
    y3 = _make_kmid(64, 32)(a2, y2, dinv, b2.reshape(1, -1), W3)
    a3 = _agg[32](y3, src, dst)
    y4 = _make_kmid(32, 2)(a3, y3, dinv, b3.reshape(1, -1), W4)
    a4 = _agg[16](y4, src, dst)
    return _k5(a4, y4, dinv, b4.reshape(1, -1), batch.reshape(-1, 1))


# speedup vs baseline: 1.0554x; 1.0554x over previous
import jax
import jax.numpy as jnp
from jax import lax
from jax.experimental import pallas as pl
from jax.experimental.pallas import tpu as pltpu
from jax.experimental.pallas import tpu_sc as plsc

N = 10000
E = 320000
G = 128
FW = 128
NC = 2
NS = 16
NW = NC * NS
CH = 128
NCH = 80
NCHH = NCH // 2
EPW = NCH * CH
EPAD = NW * EPW
NPAD = 10240
RPT = NPAD // NS


def _sc_mesh():
    return plsc.VectorSubcoreMesh(core_axis_name="c", subcore_axis_name="s",
                                  num_cores=NC, num_subcores=NS)


def _make_agg(FA):
    NSUB = FA // 16

    def body(y_hbm, src_hbm, dst_hbm, out_hbm,
             src_v, dst_v, rows_a, rows_b, wout, tbl, acc,
             sem_a, sem_b, sem_sa, sem_sb):
        c = lax.axis_index("c")
        s = lax.axis_index("s")
        wid = s * NC + c
        zeros16 = jnp.zeros((16,), jnp.float32)

        def zstep(r4, carry):
            r = r4 * 4
            for d in range(4):
                for k in range(NSUB):
                    rows_a[r + d, pl.ds(k * 16, 16)] = zeros16
            return carry

        lax.fori_loop(0, CH // 4, zstep, 0)
        for t in range(RPT // CH):
            pltpu.sync_copy(rows_a, acc.at[pl.ds(s * RPT + t * CH, CH)])
        for t in range(RPT // CH):
            pltpu.sync_copy(y_hbm.at[pl.ds(s * RPT + t * CH, CH)], wout)

            def tstep(r4, carry):
                r = r4 * 4
                for d in range(4):
                    for k in range(NSUB):
                        rows_a[r + d, pl.ds(k * 16, 16)] = (
                            wout[r + d, pl.ds(k * 16, 16)])
                return carry

            lax.fori_loop(0, CH // 4, tstep, 0)
            pltpu.sync_copy(rows_a, tbl.at[pl.ds(s * RPT + t * CH, CH)])
        plsc.subcore_barrier()

        def fire_g(j, rows, sem):
            pltpu.async_copy(tbl.at[src_v.at[j]], rows, sem)

        def wait_g(j, rows, sem):
            pltpu.make_async_copy(tbl.at[src_v.at[j]], rows, sem).wait()

        def fire_s(j, rows, sem):
            pltpu.async_copy(rows, acc.at[dst_v.at[j]], sem, add=True)

        def wait_s(j, rows, sem):
            pltpu.make_async_copy(rows, acc.at[dst_v.at[j]], sem).wait()

        def step(j2, carry):
            j = j2 * 2
            wait_g(j, rows_a, sem_a)
            fire_s(j, rows_a, sem_sa)
            wait_s(j, rows_a, sem_sa)

            @pl.when(j + 2 < NCHH)
            def _():
                fire_g(j + 2, rows_a, sem_a)

            wait_g(j + 1, rows_b, sem_b)
            fire_s(j + 1, rows_b, sem_sb)
            wait_s(j + 1, rows_b, sem_sb)

            @pl.when(j + 3 < NCHH)
            def _():
                fire_g(j + 3, rows_b, sem_b)

            return carry

        for half in range(2):
            pltpu.sync_copy(src_hbm.at[wid, pl.ds(half * NCHH, NCHH)], src_v)
            pltpu.sync_copy(dst_hbm.at[wid, pl.ds(half * NCHH, NCHH)], dst_v)
            fire_g(0, rows_a, sem_a)
            fire_g(1, rows_b, sem_b)
            lax.fori_loop(0, NCHH // 2, step, 0)

        plsc.subcore_barrier()
        for t in range(RPT // CH):
            pltpu.sync_copy(acc.at[pl.ds(s * RPT + t * CH, CH)], rows_a)

            def wstep(r4, carry):
                r = r4 * 4
                for d in range(4):
                    for k in range(NSUB):
                        wout[r + d, pl.ds(k * 16, 16)] = (
                            rows_a[r + d, pl.ds(k * 16, 16)])
                return carry

            lax.fori_loop(0, CH // 4, wstep, 0)
            pltpu.sync_copy(wout, out_hbm.at[c, pl.ds(s * RPT + t * CH, CH)])

    return pl.kernel(
        body,
        out_type=jax.ShapeDtypeStruct((NC, NPAD, FW), jnp.float32),
        mesh=_sc_mesh(),
        compiler_params=pltpu.CompilerParams(use_tc_tiling_on_sc=False),
        scratch_types=[
            pltpu.VMEM((NCHH, CH), jnp.int32),
            pltpu.VMEM((NCHH, CH), jnp.int32),
            pltpu.VMEM((CH, FA), jnp.float32),
            pltpu.VMEM((CH, FA), jnp.float32),
            pltpu.VMEM((CH, FW), jnp.float32),
            pltpu.VMEM_SHARED((NPAD, FA), jnp.float32),
            pltpu.VMEM_SHARED((NPAD, FA), jnp.float32),
            pltpu.SemaphoreType.DMA,
            pltpu.SemaphoreType.DMA,
            pltpu.SemaphoreType.DMA,
            pltpu.SemaphoreType.DMA,
        ],
    )


def _deg_body(dst_hbm, out_hbm, dst_v, ones_v, rows, pk, acc, sem_s):
    c = lax.axis_index("c")
    s = lax.axis_index("s")
    wid = s * NC + c
    zeros16 = jnp.zeros((16,), jnp.float32)
    ones16 = jnp.ones((16,), jnp.float32)

    def fill(r, carry):
        ones_v[r, pl.ds(0, 16)] = ones16
        pk[r, pl.ds(0, 16)] = zeros16
        return carry

    lax.fori_loop(0, CH, fill, 0)
    for t in range(RPT // CH):
        pltpu.sync_copy(pk, acc.at[pl.ds(s * RPT + t * CH, CH)])
    pltpu.sync_copy(dst_hbm.at[wid], dst_v)
    plsc.subcore_barrier()

    DEPTH = 4

    def fire_s(j):
        pltpu.async_copy(ones_v, acc.at[dst_v.at[j]], sem_s, add=True)

    def wait_s(j):
        pltpu.make_async_copy(ones_v, acc.at[dst_v.at[j]], sem_s).wait()

    for j in range(DEPTH):
        fire_s(j)

    def step(j, carry):
        fire_s(j)
        wait_s(j)
        return carry

    lax.fori_loop(DEPTH, NCH, step, 0)
    for j in range(DEPTH):
        wait_s(j)
    plsc.subcore_barrier()
    for t in range(RPT // CH):
        pltpu.sync_copy(acc.at[pl.ds(s * RPT + t * CH, CH)], pk)

        def wstep(r4, carry):
            r = r4 * 4
            for d in range(4):
                rows[r + d, pl.ds(0, 16)] = pk[r + d, pl.ds(0, 16)]
            return carry

        lax.fori_loop(0, CH // 4, wstep, 0)
        pltpu.sync_copy(rows, out_hbm.at[c, pl.ds(s * RPT + t * CH, CH)])


_deg = pl.kernel(
    _deg_body,
    out_type=jax.ShapeDtypeStruct((NC, NPAD, FW), jnp.float32),
    mesh=_sc_mesh(),
    compiler_params=pltpu.CompilerParams(use_tc_tiling_on_sc=False),
    scratch_types=[
        pltpu.VMEM((NCH, CH), jnp.int32),
        pltpu.VMEM((CH, 16), jnp.float32),
        pltpu.VMEM((CH, FW), jnp.float32),
        pltpu.VMEM((CH, 16), jnp.float32),
        pltpu.VMEM_SHARED((NPAD, 16), jnp.float32),
        pltpu.SemaphoreType.DMA,
    ],
)

_agg = {FA: _make_agg(FA) for FA in (64, 32, 16)}


def _pad_table(v):
    n, f = v.shape
    if f < FW:
        v = jnp.concatenate(
            [v, jnp.zeros((n, FW - f), jnp.float32)], axis=1)
    if n < NPAD:
        v = jnp.concatenate(
            [v, jnp.zeros((NPAD - n, FW), jnp.float32)], axis=0)
    return v


def _k1_body(x_ref, w_ref, degp_ref, y_ref, dinv_ref):
    deg = degp_ref[0, :N, 0:1] + degp_ref[1, :N, 0:1] + 1.0
    dinv = lax.rsqrt(deg)
    dinv_ref[...] = dinv
    y = dinv * jnp.dot(x_ref[...], w_ref[...],
                       preferred_element_type=jnp.float32)
    y_ref[...] = _pad_table(y)


def _make_kmid(f_in, f_out):
    def body(a_ref, y_ref, dinv_ref, b_ref, w_ref, o_ref):
        dinv = dinv_ref[...]
        pre = dinv * (a_ref[0, :N, :f_in] + a_ref[1, :N, :f_in]
                      + y_ref[:N, :f_in]) + b_ref[...]
        h = jnp.maximum(pre, 0.0)
        y = dinv * jnp.dot(h, w_ref[...], preferred_element_type=jnp.float32)
        o_ref[...] = _pad_table(y)

    return pl.pallas_call(
        body, out_shape=jax.ShapeDtypeStruct((NPAD, FW), jnp.float32))


def _k5_body(a_ref, y_ref, dinv_ref, b_ref, batch_ref, o_ref):
    dinv = dinv_ref[...]
    pre = dinv * (a_ref[0, :N, :2] + a_ref[1, :N, :2]
                  + y_ref[:N, :2]) + b_ref[...]
    oh = (batch_ref[...] == lax.broadcasted_iota(jnp.int32, (1, G), 1))
    oh = oh.astype(jnp.float32)
    cdims = (((0,), (0,)), ((), ()))
    sums = lax.dot_general(oh, pre, cdims, preferred_element_type=jnp.float32)
    cnts = lax.dot_general(oh, jnp.ones((N, 1), jnp.float32), cdims,
                           preferred_element_type=jnp.float32)
    pooled = sums / jnp.maximum(cnts, 1.0)
    m = jnp.max(pooled, axis=1, keepdims=True)
    o_ref[...] = pooled - m - jnp.log(
        jnp.sum(jnp.exp(pooled - m), axis=1, keepdims=True))


_k1 = pl.pallas_call(
    _k1_body,
    out_shape=[jax.ShapeDtypeStruct((NPAD, FW), jnp.float32),
               jax.ShapeDtypeStruct((N, 1), jnp.float32)],
)

_k5 = pl.pallas_call(
    _k5_body,
    out_shape=jax.ShapeDtypeStruct((G, 2), jnp.float32),
)


def kernel(x, edge_index, batch, W1, b1, W2, b2, W3, b3, W4, b4):
    npad = EPAD - E
    pad_src = (jnp.arange(npad, dtype=jnp.int32) * 37) % N
    pad_dst = N + (jnp.arange(npad, dtype=jnp.int32) % (NPAD - N))
    src = jnp.concatenate([edge_index[0], pad_src]).reshape(NW, NCH, CH)
    dst = jnp.concatenate([edge_index[1], pad_dst]).reshape(NW, NCH, CH)

    degp = _deg(dst)
    y1, dinv = _k1(x, W1, degp)
    a1 = _agg[64](y1, src, dst)
    y2 = _make_kmid(64, 64)(a1, y1, dinv, b1.reshape(1, -1), W2)
    a2 = _agg[64](y2, src, dst)
    y3 = _make_kmid(64, 32)(a2, y2, dinv, b2.reshape(1, -1), W3)
    a3 = _agg[32](y3, src, dst)
    y4 = _make_kmid(32, 2)(a3, y3, dinv, b3.reshape(1, -1), W4)
    a4 = _agg[16](y4, src, dst)
    return _k5(a4, y4, dinv, b4.reshape(1, -1), batch.reshape(-1, 1))
